# baseline (device time: 29884 ns/iter reference)
import jax
import jax.numpy as jnp
from jax import lax
from jax.experimental import pallas as pl
from jax.experimental.pallas import tpu as pltpu

N_DEV = 4


def kernel(q, k, v):
    s_per, d = q.shape
    scale = 1.0 / (d ** 0.5)

    def body(q_ref, k_ref, v_ref, o_ref, comm_ref, send_sems, recv_sems):
        my_pos = lax.axis_index("i")
        left = (my_pos - 1) % N_DEV
        right = (my_pos + 1) % N_DEV

        comm_ref[0, pl.ds(0, s_per), :] = k_ref[:, :].astype(jnp.bfloat16)
        comm_ref[0, pl.ds(s_per, s_per), :] = v_ref[:, :].astype(jnp.bfloat16)

        barrier_sem = pltpu.get_barrier_semaphore()
        for nbr in [left, right]:
            pl.semaphore_signal(
                barrier_sem, inc=1,
                device_id=(nbr,), device_id_type=pl.DeviceIdType.MESH,
            )
        pl.semaphore_wait(barrier_sem, 2)

        q_bf = q_ref[:, :].astype(jnp.bfloat16)
        m = jnp.full((s_per, 1), -1e30, dtype=jnp.float32)
        l = jnp.zeros((s_per, 1), dtype=jnp.float32)
        acc = jnp.zeros((s_per, d), dtype=jnp.float32)

        for h in range(N_DEV):
            cur = h % 2
            if h < N_DEV - 1:
                rdma = pltpu.make_async_remote_copy(
                    src_ref=comm_ref.at[cur],
                    dst_ref=comm_ref.at[(h + 1) % 2],
                    send_sem=send_sems.at[cur],
                    recv_sem=recv_sems.at[(h + 1) % 2],
                    device_id=(right,),
                    device_id_type=pl.DeviceIdType.MESH,
                )
                rdma.start()

            k_h = comm_ref[cur, pl.ds(0, s_per), :]
            v_h = comm_ref[cur, pl.ds(s_per, s_per), :]
            s = jax.lax.dot_general(
                q_bf, k_h,
                dimension_numbers=(((1,), (1,)), ((), ())),
                preferred_element_type=jnp.float32,
            ) * scale
            m_new = jnp.maximum(m, jnp.max(s, axis=1, keepdims=True))
            p = jnp.exp(s - m_new)
            alpha = jnp.exp(m - m_new)
            l = l * alpha + jnp.sum(p, axis=1, keepdims=True)
            acc = acc * alpha + jax.lax.dot_general(
                p.astype(jnp.bfloat16), v_h,
                dimension_numbers=(((1,), (0,)), ((), ())),
                preferred_element_type=jnp.float32,
            )
            m = m_new

            if h < N_DEV - 1:
                rdma.wait()

        o_ref[:, :] = acc / l

    return pl.pallas_call(
        body,
        out_shape=jax.ShapeDtypeStruct((s_per, d), jnp.float32),
        in_specs=[
            pl.BlockSpec(memory_space=pltpu.VMEM),
            pl.BlockSpec(memory_space=pltpu.VMEM),
            pl.BlockSpec(memory_space=pltpu.VMEM),
        ],
        out_specs=pl.BlockSpec(memory_space=pltpu.VMEM),
        scratch_shapes=[
            pltpu.VMEM((2, 2 * s_per, d), jnp.bfloat16),
            pltpu.SemaphoreType.DMA((2,)),
            pltpu.SemaphoreType.DMA((2,)),
        ],
        compiler_params=pltpu.CompilerParams(collective_id=0),
    )(q, k, v)


# device time: 23203 ns/iter; 1.2879x vs baseline; 1.2879x over previous
import jax
import jax.numpy as jnp
from jax import lax
from jax.experimental import pallas as pl
from jax.experimental.pallas import tpu as pltpu

N_DEV = 4


def kernel(q, k, v):
    s_per, d = q.shape
    scale = 1.0 / (d ** 0.5)

    def body(q_ref, k_ref, v_ref, o_ref, comm_ref, send_sems, recv_sems):
        my_pos = lax.axis_index("i")

        comm_ref[0, pl.ds(0, s_per), :] = k_ref[:, :].astype(jnp.bfloat16)
        comm_ref[0, pl.ds(s_per, s_per), :] = v_ref[:, :].astype(jnp.bfloat16)

        barrier_sem = pltpu.get_barrier_semaphore()
        for off in (1, 2, 3):
            pl.semaphore_signal(
                barrier_sem, inc=1,
                device_id=((my_pos + off) % N_DEV,),
                device_id_type=pl.DeviceIdType.MESH,
            )
        pl.semaphore_wait(barrier_sem, 3)

        rdmas = {}
        for off in (1, 2, 3):
            dst_slot = N_DEV - off
            rdma = pltpu.make_async_remote_copy(
                src_ref=comm_ref.at[0],
                dst_ref=comm_ref.at[dst_slot],
                send_sem=send_sems.at[off - 1],
                recv_sem=recv_sems.at[dst_slot],
                device_id=((my_pos + off) % N_DEV,),
                device_id_type=pl.DeviceIdType.MESH,
            )
            rdma.start()
            rdmas[dst_slot] = rdma

        q_bf = q_ref[:, :].astype(jnp.bfloat16)
        m = jnp.full((s_per, 1), -1e30, dtype=jnp.float32)
        l = jnp.zeros((s_per, 1), dtype=jnp.float32)
        acc = jnp.zeros((s_per, d), dtype=jnp.float32)

        def accumulate(slot, m, l, acc):
            k_h = comm_ref[slot, pl.ds(0, s_per), :]
            v_h = comm_ref[slot, pl.ds(s_per, s_per), :]
            s = lax.dot_general(
                q_bf, k_h,
                dimension_numbers=(((1,), (1,)), ((), ())),
                preferred_element_type=jnp.float32,
            ) * scale
            m_new = jnp.maximum(m, jnp.max(s, axis=1, keepdims=True))
            p = jnp.exp(s - m_new)
            alpha = jnp.exp(m - m_new)
            l = l * alpha + jnp.sum(p, axis=1, keepdims=True)
            acc = acc * alpha + lax.dot_general(
                p.astype(jnp.bfloat16), v_h,
                dimension_numbers=(((1,), (0,)), ((), ())),
                preferred_element_type=jnp.float32,
            )
            return m_new, l, acc

        m, l, acc = accumulate(0, m, l, acc)
        for slot in (3, 1, 2):
            rdmas[slot].wait_recv()
            m, l, acc = accumulate(slot, m, l, acc)

        for slot in (3, 2, 1):
            rdmas[slot].wait_send()

        o_ref[:, :] = acc / l

    return pl.pallas_call(
        body,
        out_shape=jax.ShapeDtypeStruct((s_per, d), jnp.float32),
        in_specs=[
            pl.BlockSpec(memory_space=pltpu.VMEM),
            pl.BlockSpec(memory_space=pltpu.VMEM),
            pl.BlockSpec(memory_space=pltpu.VMEM),
        ],
        out_specs=pl.BlockSpec(memory_space=pltpu.VMEM),
        scratch_shapes=[
            pltpu.VMEM((N_DEV, 2 * s_per, d), jnp.bfloat16),
            pltpu.SemaphoreType.DMA((3,)),
            pltpu.SemaphoreType.DMA((N_DEV,)),
        ],
        compiler_params=pltpu.CompilerParams(collective_id=0),
    )(q, k, v)
